# Initial kernel scaffold; baseline (speedup 1.0000x reference)
import jax
import jax.numpy as jnp
from jax.experimental import pallas as pl


def kernel(edge_index, h, p, e, snorm_n, hodge_emb, emb_h, Wp, bp, emb_e, W_pre, b_pre, W_post, b_post, W_mix, b_mix, bn_gamma, bn_beta, W_r1, b_r1, W_r2, b_r2, W_r3, b_r3):
    raise NotImplementedError("write your pallas kernel here")


# speedup vs baseline: 1.0004x; 1.0004x over previous
import jax
import jax.numpy as jnp
from jax.experimental import pallas as pl
from jax.experimental.pallas import tpu as pltpu

N = 10000
E = 320000
HID = 128
EDIM = 128
TOWERS = 4
TIN = HID // TOWERS
L = 4
AVG_D_LOG = 3.4965


def _readout_body(x_ref, w1_ref, b1_ref, w2_ref, b2_ref, w3_ref, b3_ref, o_ref):
    hg = jnp.mean(x_ref[...], axis=0, keepdims=True)
    r = jax.nn.relu(hg @ w1_ref[...] + b1_ref[...])
    r = jax.nn.relu(r @ w2_ref[...] + b2_ref[...])
    o_ref[...] = r @ w3_ref[...] + b3_ref[...]


def kernel(edge_index, h, p, e, snorm_n, hodge_emb, emb_h, Wp, bp, emb_e, W_pre, b_pre,
           W_post, b_post, W_mix, b_mix, bn_gamma, bn_beta, W_r1, b_r1, W_r2, b_r2, W_r3, b_r3):
    del hodge_emb
    src = edge_index[0]
    dst = edge_index[1]
    x = jnp.take(emb_h, h, axis=0)
    x = x + (p @ Wp + bp)
    ef = jnp.take(emb_e, e, axis=0)
    deg = jax.ops.segment_sum(jnp.ones((E,), jnp.float32), dst, num_segments=N)
    degc = jnp.maximum(deg, 1.0)
    log_deg = jnp.log(degc + 1.0)
    amp = (log_deg / AVG_D_LOG)[:, None]
    att = (AVG_D_LOG / log_deg)[:, None]
    has = (deg > 0)[:, None]
    for l in range(L):
        h_in = x
        touts = []
        for t in range(TOWERS):
            ht = x[:, t * TIN:(t + 1) * TIN]
            m_in = jnp.concatenate([ht[src], ht[dst], ef], axis=1)
            msg = jax.nn.relu(m_in @ W_pre[l, t] + b_pre[l, t])
            mean = jax.ops.segment_sum(msg, dst, num_segments=N) / degc[:, None]
            sq = jax.ops.segment_sum(msg * msg, dst, num_segments=N) / degc[:, None]
            std = jnp.sqrt(jax.nn.relu(sq - mean * mean) + 1e-5)
            mx = jnp.where(has, jax.ops.segment_max(msg, dst, num_segments=N), 0.0)
            mn = jnp.where(has, jax.ops.segment_min(msg, dst, num_segments=N), 0.0)
            feats = [ht]
            for a in (mean, mx, mn, std):
                feats.extend([a, a * amp, a * att])
            post_in = jnp.concatenate(feats, axis=1)
            touts.append(post_in @ W_post[l, t] + b_post[l, t])
        hcat = jnp.concatenate(touts, axis=1)
        hmix = jax.nn.leaky_relu(hcat @ W_mix[l] + b_mix[l])
        hn = hmix * snorm_n
        mu = jnp.mean(hn, axis=0, keepdims=True)
        var = jnp.var(hn, axis=0, keepdims=True)
        hn = (hn - mu) / jnp.sqrt(var + 1e-5) * bn_gamma[l] + bn_beta[l]
        hn = jax.nn.relu(hn)
        x = h_in + hn
    out = pl.pallas_call(
        _readout_body,
        out_shape=jax.ShapeDtypeStruct((1, 1), jnp.float32),
    )(x, W_r1, b_r1[None, :], W_r2, b_r2[None, :], W_r3, b_r3[None, :])
    return out[0]


# speedup vs baseline: 7.2385x; 7.2356x over previous
import functools
import jax
import jax.numpy as jnp
from jax import lax
from jax.experimental import pallas as pl
from jax.experimental.pallas import tpu as pltpu
from jax.experimental.pallas import tpu_sc as plsc

N = 10000
E = 320000
HID = 128
EDIM = 128
TOWERS = 4
TIN = HID // TOWERS
L = 4
AVG_D_LOG = 3.4965

NTILES = 32
NPT = 320
N_PAD = NTILES * NPT
C_IN = 2048
E_PAD = ((E + C_IN - 1) // C_IN) * C_IN
STAGE = 4112
FLUSH = 2048
CAP_ROW = E_PAD + 4096
K_E = 128
BIG = 3.0e38
NB = 20
RB = N_PAD // NB

_mesh = plsc.VectorSubcoreMesh(core_axis_name="c", subcore_axis_name="s")


def _wid():
    return lax.axis_index("s") * 2 + lax.axis_index("c")


def _m8(x):
    return pl.multiple_of(x, 8)


@functools.partial(
    pl.kernel, mesh=_mesh,
    compiler_params=pltpu.CompilerParams(needs_layout_passes=False),
    out_type=[
        jax.ShapeDtypeStruct((NTILES * CAP_ROW,), jnp.int32),
        jax.ShapeDtypeStruct((NTILES * CAP_ROW,), jnp.int32),
        jax.ShapeDtypeStruct((NTILES * CAP_ROW,), jnp.int32),
        jax.ShapeDtypeStruct((NTILES * 16,), jnp.int32),
    ],
    scratch_types=[
        pltpu.VMEM((C_IN,), jnp.int32),
        pltpu.VMEM((C_IN,), jnp.int32),
        pltpu.VMEM((C_IN,), jnp.int32),
        pltpu.VMEM((STAGE,), jnp.int32),
        pltpu.VMEM((STAGE,), jnp.int32),
        pltpu.VMEM((STAGE,), jnp.int32),
        pltpu.VMEM((16,), jnp.int32),
        pltpu.SemaphoreType.DMA,
    ],
)
def _bucket(dst_h, src_h, et_h, osrc, oet, odstl, ocnt,
            dbuf, sbuf, ebuf, st_s, st_e, st_d, cntb, sem):
    w = _wid()
    lo = w * NPT
    zi = jnp.zeros((16,), jnp.int32)
    for i in range(STAGE // 16):
        st_s[pl.ds(i * 16, 16)] = zi
        st_e[pl.ds(i * 16, 16)] = zi
        st_d[pl.ds(i * 16, 16)] = zi

    def chunk_body(ci, carry):
        written, pos = carry
        pltpu.sync_copy(dst_h.at[pl.ds(_m8(ci * C_IN), C_IN)], dbuf)
        pltpu.sync_copy(src_h.at[pl.ds(_m8(ci * C_IN), C_IN)], sbuf)
        pltpu.sync_copy(et_h.at[pl.ds(_m8(ci * C_IN), C_IN)], ebuf)

        def g_body(g, pos):
            dv = dbuf[pl.ds(g * 16, 16)]
            sv = sbuf[pl.ds(g * 16, 16)]
            ev = ebuf[pl.ds(g * 16, 16)]
            mask = (dv >= lo) & (dv < lo + NPT)
            mi = jnp.where(mask, 1, 0).astype(jnp.int32)
            lane = lax.iota(jnp.int32, 16)
            cum = mi
            for d in (1, 2, 4, 8):
                sh = cum.at[jnp.maximum(lane - d, 0)].get(
                    mode='promise_in_bounds')
                cum = cum + jnp.where(lane >= d, sh, 0)
            posv = pos + cum - mi
            plsc.store_scatter(st_s, [posv], sv, mask=mask)
            plsc.store_scatter(st_e, [posv], ev, mask=mask)
            plsc.store_scatter(st_d, [posv], dv - lo, mask=mask)
            return pos + cum[15]

        pos = lax.fori_loop(0, C_IN // 16, g_body, pos)

        def do_flush(carry):
            written, pos = carry
            pltpu.sync_copy(st_s.at[pl.ds(0, FLUSH)],
                            osrc.at[pl.ds(_m8(w * CAP_ROW + written), FLUSH)])
            pltpu.sync_copy(st_e.at[pl.ds(0, FLUSH)],
                            oet.at[pl.ds(_m8(w * CAP_ROW + written), FLUSH)])
            pltpu.sync_copy(st_d.at[pl.ds(0, FLUSH)],
                            odstl.at[pl.ds(_m8(w * CAP_ROW + written), FLUSH)])
            for k in range(FLUSH // 16):
                st_s[pl.ds(k * 16, 16)] = st_s[pl.ds(FLUSH + k * 16, 16)]
                st_e[pl.ds(k * 16, 16)] = st_e[pl.ds(FLUSH + k * 16, 16)]
                st_d[pl.ds(k * 16, 16)] = st_d[pl.ds(FLUSH + k * 16, 16)]
            return written + FLUSH, pos - FLUSH

        return lax.cond(pos >= FLUSH, do_flush, lambda c: c, (written, pos))

    written, pos = lax.fori_loop(0, E_PAD // C_IN, chunk_body, (0, 0))
    pltpu.sync_copy(st_s.at[pl.ds(0, 4096)],
                    osrc.at[pl.ds(_m8(w * CAP_ROW + written), 4096)])
    pltpu.sync_copy(st_e.at[pl.ds(0, 4096)],
                    oet.at[pl.ds(_m8(w * CAP_ROW + written), 4096)])
    pltpu.sync_copy(st_d.at[pl.ds(0, 4096)],
                    odstl.at[pl.ds(_m8(w * CAP_ROW + written), 4096)])
    cntb[pl.ds(0, 16)] = jnp.full((16,), written + pos, jnp.int32)
    pltpu.sync_copy(cntb, ocnt.at[pl.ds(_m8(w * 16), 16)])


@functools.partial(
    pl.kernel, mesh=_mesh,
    compiler_params=pltpu.CompilerParams(needs_layout_passes=False),
    out_type=[
        jax.ShapeDtypeStruct((N_PAD * 256,), jnp.float32),
        jax.ShapeDtypeStruct((N_PAD * 16,), jnp.float32),
    ],
    scratch_types=[
        pltpu.VMEM(((NPT + 1) * 256,), jnp.float32),
        pltpu.VMEM(((NPT + 1) * 16,), jnp.float32),
        pltpu.VMEM(((NPT + 1) * 64,), jnp.float32),
        pltpu.VMEM((256,), jnp.float32),
        pltpu.VMEM((K_E,), jnp.int32),
        pltpu.VMEM((K_E,), jnp.int32),
        pltpu.VMEM((K_E,), jnp.int32),
        pltpu.VMEM((K_E,), jnp.int32),
        pltpu.VMEM((K_E, 128), jnp.float32),
        pltpu.VMEM((16,), jnp.int32),
        pltpu.SemaphoreType.DMA,
    ],
)
def _edge_pass(a_h, b_h, ctab_h, lsrc, let, ldst, cnt_h, ost, odeg,
               acc, dacc, bl, ctl, sbuf, sbuf2, ebuf, dbuf, rows, cb, sem):
    w = _wid()
    zf = jnp.zeros((16,), jnp.float32)
    bigf = jnp.full((16,), BIG, jnp.float32)
    one = jnp.ones((16,), jnp.float32)
    for i in range(NPT + 1):
        for k in range(12):
            acc[pl.ds(i * 256 + k * 16, 16)] = zf
        for k in range(4):
            acc[pl.ds(i * 256 + 192 + k * 16, 16)] = bigf
        dacc[pl.ds(i * 16, 16)] = zf
    pltpu.sync_copy(b_h.at[pl.ds(_m8(w * NPT * 64), NPT * 64)],
                    bl.at[pl.ds(0, NPT * 64)])
    pltpu.sync_copy(ctab_h, ctl)
    pltpu.sync_copy(cnt_h.at[pl.ds(_m8(w * 16), 16)], cb)
    cnt = cb[pl.ds(0, 16)][0]
    nch = (cnt + K_E - 1) // K_E

    def chunk_body(ci, _):
        pltpu.sync_copy(lsrc.at[pl.ds(_m8(w * CAP_ROW + ci * K_E), K_E)], sbuf)
        pltpu.sync_copy(let.at[pl.ds(_m8(w * CAP_ROW + ci * K_E), K_E)], ebuf)
        pltpu.sync_copy(ldst.at[pl.ds(_m8(w * CAP_ROW + ci * K_E), K_E)], dbuf)
        for g in range(K_E // 16):
            sbuf2[pl.ds(g * 16, 16)] = lax.shift_right_logical(
                sbuf[pl.ds(g * 16, 16)], 1)
        pltpu.async_copy(a_h.at[sbuf2], rows, sem).wait()
        rem = cnt - ci * K_E

        def g_body(g, _):
            dvec = dbuf[pl.ds(g * 16, 16)]
            evec = ebuf[pl.ds(g * 16, 16)]
            svec = sbuf[pl.ds(g * 16, 16)]
            for j in range(16):
                valid = g * 16 + j < rem
                dl = jnp.where(valid, dvec[j], NPT)
                et = jnp.where(valid, evec[j], 0)
                co = (svec[j] & 1) * 64
                base = dl * 256
                bb = dl * 64
                eb = et * 64
                i = g * 16 + j
                for k in range(4):
                    a = rows[i, pl.ds(co + k * 16, 16)]
                    b = bl[pl.ds(bb + k * 16, 16)]
                    c = ctl[pl.ds(eb + k * 16, 16)]
                    m = jnp.maximum(a + b + c, 0.0)
                    o = base + k * 16
                    acc[pl.ds(o, 16)] = acc[pl.ds(o, 16)] + m
                    acc[pl.ds(o + 64, 16)] = acc[pl.ds(o + 64, 16)] + m * m
                    acc[pl.ds(o + 128, 16)] = jnp.maximum(
                        acc[pl.ds(o + 128, 16)], m)
                    acc[pl.ds(o + 192, 16)] = jnp.minimum(
                        acc[pl.ds(o + 192, 16)], m)
                db = dl * 16
                dacc[pl.ds(db, 16)] = dacc[pl.ds(db, 16)] + one
            return 0

        lax.fori_loop(0, K_E // 16, g_body, 0)
        return 0

    lax.fori_loop(0, nch, chunk_body, 0)
    pltpu.sync_copy(acc.at[pl.ds(0, NPT * 256)],
                    ost.at[pl.ds(_m8(w * NPT * 256), NPT * 256)])
    pltpu.sync_copy(dacc.at[pl.ds(0, NPT * 16)],
                    odeg.at[pl.ds(_m8(w * NPT * 16), NPT * 16)])


def _embed_body(h_ref, p_ref, eh_ref, wp_ref, bp_ref, p1_ref, p2_ref,
                ox, oa0, oa1, ob0, ob1):
    hv = h_ref[...]
    oh = (hv == lax.broadcasted_iota(jnp.int32, (RB, 32), 1)).astype(jnp.float32)
    x = oh @ eh_ref[...] + p_ref[...] @ wp_ref[...] + bp_ref[...]
    a = x @ p1_ref[...]
    b = x @ p2_ref[...]
    ox[...] = x
    oa0[...] = a[:, :64]
    oa1[...] = a[:, 64:]
    ob0[...] = b[:, :64]
    ob1[...] = b[:, 64:]


def _proj_body(hin_ref, hraw_ref, ps_ref, pq_ref, g_ref, bta_ref, p1_ref, p2_ref,
               ox, oa0, oa1, ob0, ob1):
    s = jnp.sum(ps_ref[...], axis=0, keepdims=True) / (8 * N)
    q = jnp.sum(pq_ref[...], axis=0, keepdims=True) / (8 * N)
    var = q - s * s
    inv = lax.rsqrt(var + 1e-5)
    s1 = g_ref[...] * inv
    s2 = bta_ref[...] - s * s1
    hn = jnp.maximum(hraw_ref[...] * s1 + s2, 0.0)
    x = hin_ref[...] + hn
    a = x @ p1_ref[...]
    b = x @ p2_ref[...]
    ox[...] = x
    oa0[...] = a[:, :64]
    oa1[...] = a[:, 64:]
    ob0[...] = b[:, :64]
    ob1[...] = b[:, 64:]


def _post_body(x_ref, st0_ref, st1_ref, deg_ref, sn_ref, w13_ref, b13_ref,
               wm_ref, bm_ref, ohraw, ops, opq):
    deg = deg_ref[...]
    degc = jnp.maximum(deg, 1.0)
    ld = jnp.log(degc + 1.0)
    amp = ld / AVG_D_LOG
    att = AVG_D_LOG / ld
    has = deg > 0.0
    st0 = st0_ref[...]
    st1 = st1_ref[...]
    S = jnp.concatenate([st0[:, 0:64], st1[:, 0:64]], axis=1)
    Q = jnp.concatenate([st0[:, 64:128], st1[:, 64:128]], axis=1)
    MX = jnp.concatenate([st0[:, 128:192], st1[:, 128:192]], axis=1)
    MN = jnp.concatenate([st0[:, 192:256], st1[:, 192:256]], axis=1)
    mean = S / degc
    msq = Q / degc
    std = jnp.sqrt(jnp.maximum(msq - mean * mean, 0.0) + 1e-5)
    mx = jnp.where(has, MX, 0.0)
    mn = jnp.where(has, MN, 0.0)
    x = x_ref[...]
    feats = (x, mean, mean * amp, mean * att, mx, mx * amp, mx * att,
             mn, mn * amp, mn * att, std, std * amp, std * att)
    hcat = b13_ref[...]
    for j, f in enumerate(feats):
        hcat = hcat + f @ w13_ref[j]
    hm = hcat @ wm_ref[...] + bm_ref[...]
    hm = jnp.where(hm > 0, hm, 0.01 * hm)
    hraw = hm * sn_ref[...]
    ohraw[...] = hraw
    ops[...] = jnp.broadcast_to(jnp.sum(hraw, axis=0, keepdims=True),
                                (8, HID))[None]
    opq[...] = jnp.broadcast_to(jnp.sum(hraw * hraw, axis=0, keepdims=True),
                                (8, HID))[None]


def _final_body(hin_ref, hraw_ref, ps_ref, pq_ref, g_ref, bta_ref, ohg):
    s = jnp.sum(ps_ref[...], axis=0, keepdims=True) / (8 * N)
    q = jnp.sum(pq_ref[...], axis=0, keepdims=True) / (8 * N)
    inv = lax.rsqrt(q - s * s + 1e-5)
    s1 = g_ref[...] * inv
    s2 = bta_ref[...] - s * s1
    hn = jnp.maximum(hraw_ref[...] * s1 + s2, 0.0)
    x = hin_ref[...] + hn
    rid = (pl.program_id(0) * RB
           + lax.broadcasted_iota(jnp.int32, (RB, 1), 0))
    x = jnp.where(rid < N, x, 0.0)
    ohg[...] = jnp.broadcast_to(jnp.sum(x, axis=0, keepdims=True),
                                (8, HID))[None]


def _readout_body(hgp_ref, w1_ref, b1_ref, w2_ref, b2_ref, w3_ref, b3_ref, o_ref):
    hg = jnp.sum(hgp_ref[...], axis=0, keepdims=True) / (8 * N)
    r = jnp.maximum(hg @ w1_ref[...] + b1_ref[...], 0.0)
    r = jnp.maximum(r @ w2_ref[...] + b2_ref[...], 0.0)
    o_ref[...] = r @ w3_ref[...] + b3_ref[...]


def _full(shape):
    return pl.BlockSpec(shape, lambda i: tuple(0 for _ in shape))


def _rows(c):
    return pl.BlockSpec((RB, c), lambda i: (i, 0))


_embed_call = pl.pallas_call(
    _embed_body, grid=(NB,),
    in_specs=[_rows(1), _rows(8), _full((32, HID)), _full((8, HID)),
              _full((1, HID)), _full((HID, HID)), _full((HID, HID))],
    out_specs=[_rows(HID), _rows(64), _rows(64), _rows(64), _rows(64)],
    out_shape=[jax.ShapeDtypeStruct((N_PAD, HID), jnp.float32)] +
              [jax.ShapeDtypeStruct((N_PAD, 64), jnp.float32)] * 4,
)

_proj_call = pl.pallas_call(
    _proj_body, grid=(NB,),
    in_specs=[_rows(HID), _rows(HID), _full((NB * 8, HID)), _full((NB * 8, HID)),
              _full((1, HID)), _full((1, HID)), _full((HID, HID)),
              _full((HID, HID))],
    out_specs=[_rows(HID), _rows(64), _rows(64), _rows(64), _rows(64)],
    out_shape=[jax.ShapeDtypeStruct((N_PAD, HID), jnp.float32)] +
              [jax.ShapeDtypeStruct((N_PAD, 64), jnp.float32)] * 4,
)

_post_call = pl.pallas_call(
    _post_body, grid=(NB,),
    in_specs=[_rows(HID), _rows(256), _rows(256), _rows(1), _rows(1),
              _full((13, HID, HID)), _full((1, HID)), _full((HID, HID)),
              _full((1, HID))],
    out_specs=[_rows(HID), pl.BlockSpec((1, 8, HID), lambda i: (i, 0, 0)),
               pl.BlockSpec((1, 8, HID), lambda i: (i, 0, 0))],
    out_shape=[jax.ShapeDtypeStruct((N_PAD, HID), jnp.float32),
               jax.ShapeDtypeStruct((NB, 8, HID), jnp.float32),
               jax.ShapeDtypeStruct((NB, 8, HID), jnp.float32)],
)

_final_call = pl.pallas_call(
    _final_body, grid=(NB,),
    in_specs=[_rows(HID), _rows(HID), _full((NB * 8, HID)), _full((NB * 8, HID)),
              _full((1, HID)), _full((1, HID))],
    out_specs=[pl.BlockSpec((1, 8, HID), lambda i: (i, 0, 0))],
    out_shape=[jax.ShapeDtypeStruct((NB, 8, HID), jnp.float32)],
)

_readout_call = pl.pallas_call(
    _readout_body,
    out_shape=jax.ShapeDtypeStruct((1, 1), jnp.float32),
)


def kernel(edge_index, h, p, e, snorm_n, hodge_emb, emb_h, Wp, bp, emb_e, W_pre, b_pre,
           W_post, b_post, W_mix, b_mix, bn_gamma, bn_beta, W_r1, b_r1, W_r2, b_r2, W_r3, b_r3):
    del hodge_emb
    f32 = jnp.float32
    src = edge_index[0].astype(jnp.int32)
    dst = edge_index[1].astype(jnp.int32)
    et = e.astype(jnp.int32)
    pad_e = E_PAD - E
    dst_p = jnp.pad(dst, (0, pad_e), constant_values=N_PAD + 7)
    src_p = jnp.pad(src, (0, pad_e))
    et_p = jnp.pad(et, (0, pad_e))

    W1 = W_pre[:, :, 0:TIN, :]
    W2 = W_pre[:, :, TIN:2 * TIN, :]
    W3 = W_pre[:, :, 2 * TIN:, :]
    eyeT = jnp.eye(TOWERS, dtype=f32)
    P1 = jnp.einsum('ltij,tu->ltiuj', W1, eyeT).reshape(L, HID, HID)
    P2 = jnp.einsum('ltij,tu->ltiuj', W2, eyeT).reshape(L, HID, HID)
    ctab = (jnp.einsum('be,lteo->ltbo', emb_e, W3)
            + b_pre[:, :, None, :]).transpose(0, 2, 1, 3).reshape(L, 4, HID)
    W13 = jnp.einsum('ltjio,tu->ljtiuo',
                     W_post.reshape(L, TOWERS, 13, TIN, TIN),
                     eyeT).reshape(L, 13, HID, HID)
    b13 = b_post.reshape(L, HID)
    emb_h_pad = jnp.pad(emb_h, ((0, 32 - emb_h.shape[0]), (0, 0)))
    h_p = jnp.pad(h.astype(jnp.int32), (0, N_PAD - N)).reshape(N_PAD, 1)
    p_p = jnp.pad(p, ((0, N_PAD - N), (0, 0)))
    sn_p = jnp.pad(snorm_n, ((0, N_PAD - N), (0, 0)))

    lsrc, let_, ldst, cnts = _bucket(dst_p, src_p, et_p)

    x, a0, a1, b0, b1 = _embed_call(h_p, p_p, emb_h_pad, Wp, bp.reshape(1, HID),
                                    P1[0], P2[0])
    hraw = ps = pq = None
    for l in range(L):
        if l > 0:
            x, a0, a1, b0, b1 = _proj_call(
                x, hraw, ps.reshape(NB * 8, HID), pq.reshape(NB * 8, HID),
                bn_gamma[l - 1].reshape(1, HID), bn_beta[l - 1].reshape(1, HID),
                P1[l], P2[l])
        st0, deg16 = _edge_pass(a0.reshape(N_PAD // 2, 128), b0.reshape(-1),
                                ctab[l, :, 0:64].reshape(-1),
                                lsrc, let_, ldst, cnts)
        st1, _ = _edge_pass(a1.reshape(N_PAD // 2, 128), b1.reshape(-1),
                            ctab[l, :, 64:128].reshape(-1),
                            lsrc, let_, ldst, cnts)
        if l == 0:
            degf = deg16.reshape(N_PAD, 16)[:, 0:1]
        hraw, ps, pq = _post_call(x, st0.reshape(N_PAD, 256),
                                  st1.reshape(N_PAD, 256), degf, sn_p,
                                  W13[l], b13[l].reshape(1, HID),
                                  W_mix[l], b_mix[l].reshape(1, HID))
    hgp = _final_call(x, hraw, ps.reshape(NB * 8, HID),
                      pq.reshape(NB * 8, HID), bn_gamma[L - 1].reshape(1, HID),
                      bn_beta[L - 1].reshape(1, HID))[0]
    hgp = hgp.reshape(NB * 8, HID)
    out = _readout_call(hgp, W_r1, b_r1.reshape(1, -1), W_r2,
                        b_r2.reshape(1, -1), W_r3, b_r3.reshape(1, 1))
    return out[0]
